# baseline (speedup 1.0000x reference)
import functools

import jax
import jax.numpy as jnp
from jax import lax
from jax.experimental import pallas as pl
from jax.experimental.pallas import tpu as pltpu
from jax.experimental.pallas import tpu_sc as plsc

N_EMBED = 768
HIDDEN = 3072
E = 64
S = 2048
TB = 64
NB = S // TB + E
NPAD = NB * TB
GS = 512

_NC = 2
_NS = 16
_NW = _NC * _NS
_CH = 96


def _router_body(x_ref, wcat_ref, bcat_ref, noise_ref,
                 dest_ref, be_ref, used_ref):
    x = x_ref[...]
    both = jnp.dot(x, wcat_ref[...], preferred_element_type=jnp.float32)
    both = both + bcat_ref[...]
    logits = both[:, :E]
    nlog = both[:, E:]
    noisy = logits + noise_ref[...] * jax.nn.softplus(nlog)
    m = jnp.max(noisy, axis=1, keepdims=True)
    col = lax.broadcasted_iota(jnp.int32, (S, E), 1)
    e_idx = jnp.min(jnp.where(noisy >= m, col, E), axis=1, keepdims=True)

    oh = (col == e_idx).astype(jnp.float32)
    counts = jnp.sum(oh, axis=0, keepdims=True)
    nblk = jnp.floor((counts + (TB - 1)) * (1.0 / TB))
    ei = lax.broadcasted_iota(jnp.int32, (E, E), 0)
    ej = lax.broadcasted_iota(jnp.int32, (E, E), 1)
    upper = (ei <= ej).astype(jnp.float32)
    blk_end = jnp.dot(nblk, upper, preferred_element_type=jnp.float32)
    pad_off = (blk_end - nblk) * TB
    used_f = blk_end[:, E - 1:E]

    gi = lax.broadcasted_iota(jnp.int32, (GS, GS), 0)
    gj = lax.broadcasted_iota(jnp.int32, (GS, GS), 1)
    tri = (gj <= gi).astype(jnp.float32)
    base = jnp.zeros((1, E), jnp.float32)
    for g in range(S // GS):
        oh_g = oh[g * GS:(g + 1) * GS, :]
        cum_g = jnp.dot(tri, oh_g, preferred_element_type=jnp.float32) + base
        dest_g = jnp.sum((cum_g - 1.0 + pad_off) * oh_g, axis=1, keepdims=True)
        dest_ref[g * GS:(g + 1) * GS, :] = dest_g.astype(jnp.int32)
        base = cum_g[GS - 1:GS, :]

    bi = lax.broadcasted_iota(jnp.int32, (NB, 1), 0).astype(jnp.float32)
    be_raw = jnp.sum((blk_end <= bi).astype(jnp.float32), axis=1, keepdims=True)
    be_clamped = jnp.minimum(be_raw, E - 1)
    last_e = jnp.sum(jnp.where(bi == used_f - 1.0, be_clamped, 0.0),
                     axis=0, keepdims=True)
    be = jnp.where(bi < used_f, be_clamped, last_e)
    be_ref[...] = be.astype(jnp.int32)
    used_ref[...] = used_f.astype(jnp.int32)


def _router(x2, wcat, bcat, noise):
    return pl.pallas_call(
        _router_body,
        out_shape=(
            jax.ShapeDtypeStruct((S, 1), jnp.int32),
            jax.ShapeDtypeStruct((NB, 1), jnp.int32),
            jax.ShapeDtypeStruct((1, 1), jnp.int32),
        ),
    )(x2, wcat, bcat, noise)


def _scatter_rows(rows, idx, n_out):
    r_in = idx.shape[0]
    per_w = r_in // _NW
    mesh = plsc.VectorSubcoreMesh(core_axis_name="c", subcore_axis_name="s")

    @functools.partial(
        pl.kernel,
        out_type=jax.ShapeDtypeStruct((n_out, N_EMBED), jnp.float32),
        mesh=mesh,
        scratch_types=[
            pltpu.VMEM((per_w,), jnp.int32),
            pltpu.VMEM((per_w, N_EMBED), jnp.float32),
            pltpu.SemaphoreType.DMA,
        ],
    )
    def sk(rows_hbm, idx_hbm, out_hbm, idx_v, rows_v, sem):
        wid = lax.axis_index("s") * _NC + lax.axis_index("c")
        base = wid * per_w
        pltpu.sync_copy(idx_hbm.at[pl.ds(base, per_w)], idx_v)
        pltpu.sync_copy(rows_hbm.at[pl.ds(base, per_w)], rows_v)
        pltpu.async_copy(rows_v, out_hbm.at[idx_v], sem).wait()

    return sk(rows, idx)


def _gather_rows(table, idx):
    r_out = idx.shape[0]
    per_w = r_out // _NW
    ch = min(per_w, _CH)
    chunks = per_w // ch
    mesh = plsc.VectorSubcoreMesh(core_axis_name="c", subcore_axis_name="s")

    @functools.partial(
        pl.kernel,
        out_type=jax.ShapeDtypeStruct((r_out, N_EMBED), jnp.float32),
        mesh=mesh,
        scratch_types=[
            pltpu.VMEM((ch,), jnp.int32),
            pltpu.VMEM((ch, N_EMBED), jnp.float32),
            pltpu.SemaphoreType.DMA,
        ],
    )
    def gk(table_hbm, idx_hbm, out_hbm, idx_v, rows_v, sem):
        wid = lax.axis_index("s") * _NC + lax.axis_index("c")
        for c in range(chunks):
            base = wid * per_w + c * ch
            pltpu.sync_copy(idx_hbm.at[pl.ds(base, ch)], idx_v)
            pltpu.async_copy(table_hbm.at[idx_v], rows_v, sem).wait()
            pltpu.sync_copy(rows_v, out_hbm.at[pl.ds(base, ch)])

    return gk(table, idx)


def _ffn_body(be_ref, used_ref, x_ref, w1_ref, b1_ref, w2_ref, b2_ref, o_ref):
    b = pl.program_id(0)

    @pl.when(b < used_ref[0])
    def _():
        x = x_ref[...]
        h = jnp.dot(x, w1_ref[0], preferred_element_type=jnp.float32)
        h = jnp.maximum(h + b1_ref[0], 0.0)
        o_ref[...] = (
            jnp.dot(h, w2_ref[0], preferred_element_type=jnp.float32)
            + b2_ref[0])


def _ffn(xp, w1, b1, w2, b2, block_expert, used):
    grid_spec = pltpu.PrefetchScalarGridSpec(
        num_scalar_prefetch=2,
        grid=(NB,),
        in_specs=[
            pl.BlockSpec((TB, N_EMBED),
                         lambda b, be, u: (jnp.minimum(b, u[0] - 1), 0)),
            pl.BlockSpec((1, N_EMBED, HIDDEN), lambda b, be, u: (be[b], 0, 0)),
            pl.BlockSpec((1, 1, HIDDEN), lambda b, be, u: (be[b], 0, 0)),
            pl.BlockSpec((1, HIDDEN, N_EMBED), lambda b, be, u: (be[b], 0, 0)),
            pl.BlockSpec((1, 1, N_EMBED), lambda b, be, u: (be[b], 0, 0)),
        ],
        out_specs=pl.BlockSpec((TB, N_EMBED),
                               lambda b, be, u: (jnp.minimum(b, u[0] - 1), 0)),
    )
    return pl.pallas_call(
        _ffn_body,
        grid_spec=grid_spec,
        out_shape=jax.ShapeDtypeStruct((NPAD, N_EMBED), jnp.float32),
        compiler_params=pltpu.CompilerParams(
            dimension_semantics=("arbitrary",),
        ),
    )(block_expert, used, xp, w1,
      b1.reshape(E, 1, HIDDEN), w2, b2.reshape(E, 1, N_EMBED))


def kernel(x, router_w, router_b, noise_w, noise_b,
           expert_w1, expert_b1, expert_w2, expert_b2):
    x2 = x.reshape(S, N_EMBED)
    wcat = jnp.concatenate([router_w, noise_w], axis=1)
    bcat = jnp.concatenate([router_b, noise_b]).reshape(1, 2 * E)
    noise = jax.random.normal(
        jax.random.key(42), (1, S, E), dtype=jnp.float32).reshape(S, E)

    dest2, be2, used2 = _router(x2, wcat, bcat, noise)
    dest = dest2[:, 0]
    xp = _scatter_rows(x2, dest, NPAD)
    yp = _ffn(xp, expert_w1, expert_b1, expert_w2, expert_b2,
              be2[:, 0], used2[0])
    y = _gather_rows(yp, dest)
    return y.reshape(x.shape)


# speedup vs baseline: 1.0070x; 1.0070x over previous
import functools

import jax
import jax.numpy as jnp
from jax import lax
from jax.experimental import pallas as pl
from jax.experimental.pallas import tpu as pltpu
from jax.experimental.pallas import tpu_sc as plsc

N_EMBED = 768
HIDDEN = 3072
E = 64
S = 2048
TB = 64
NB = S // TB + E
NPAD = NB * TB
GS = 512

_NC = 2
_NS = 16
_NW = _NC * _NS
_CH = 96


def _router_body(x_ref, wcat_ref, bcat_ref, noise_ref,
                 dest_ref, be_ref, used_ref):
    x = x_ref[...]
    both = jnp.dot(x, wcat_ref[...], preferred_element_type=jnp.float32)
    both = both + bcat_ref[...]
    logits = both[:, :E]
    nlog = both[:, E:]
    noisy = logits + noise_ref[...] * jax.nn.softplus(nlog)
    m = jnp.max(noisy, axis=1, keepdims=True)
    col = lax.broadcasted_iota(jnp.int32, (S, E), 1)
    e_idx = jnp.min(jnp.where(noisy >= m, col, E), axis=1, keepdims=True)

    oh = (col == e_idx).astype(jnp.float32)
    counts = jnp.sum(oh, axis=0, keepdims=True)
    nblk = jnp.floor((counts + (TB - 1)) * (1.0 / TB))
    ei = lax.broadcasted_iota(jnp.int32, (E, E), 0)
    ej = lax.broadcasted_iota(jnp.int32, (E, E), 1)
    upper = (ei <= ej).astype(jnp.float32)
    blk_end = jnp.dot(nblk, upper, preferred_element_type=jnp.float32)
    pad_off = (blk_end - nblk) * TB
    used_f = blk_end[:, E - 1:E]

    gi = lax.broadcasted_iota(jnp.int32, (GS, GS), 0)
    gj = lax.broadcasted_iota(jnp.int32, (GS, GS), 1)
    tri = (gj <= gi).astype(jnp.float32)
    base = jnp.zeros((1, E), jnp.float32)
    for g in range(S // GS):
        oh_g = oh[g * GS:(g + 1) * GS, :]
        cum_g = jnp.dot(tri, oh_g, preferred_element_type=jnp.float32) + base
        dest_g = jnp.sum((cum_g - 1.0 + pad_off) * oh_g, axis=1, keepdims=True)
        dest_ref[g * GS:(g + 1) * GS, :] = dest_g.astype(jnp.int32)
        base = cum_g[GS - 1:GS, :]

    bi = lax.broadcasted_iota(jnp.int32, (NB, 1), 0).astype(jnp.float32)
    be_raw = jnp.sum((blk_end <= bi).astype(jnp.float32), axis=1, keepdims=True)
    be_clamped = jnp.minimum(be_raw, E - 1)
    last_e = jnp.sum(jnp.where(bi == used_f - 1.0, be_clamped, 0.0),
                     axis=0, keepdims=True)
    be = jnp.where(bi < used_f, be_clamped, last_e)
    be_ref[...] = be.astype(jnp.int32)
    used_ref[...] = used_f.astype(jnp.int32)


def _router(x2, wcat, bcat, noise):
    return pl.pallas_call(
        _router_body,
        out_shape=(
            jax.ShapeDtypeStruct((S, 1), jnp.int32),
            jax.ShapeDtypeStruct((NB, 1), jnp.int32),
            jax.ShapeDtypeStruct((1, 1), jnp.int32),
        ),
    )(x2, wcat, bcat, noise)


def _scatter_rows(rows, idx, n_out):
    r_in = idx.shape[0]
    per_w = r_in // _NW
    mesh = plsc.VectorSubcoreMesh(core_axis_name="c", subcore_axis_name="s")

    @functools.partial(
        pl.kernel,
        out_type=jax.ShapeDtypeStruct((n_out, N_EMBED), jnp.float32),
        mesh=mesh,
        scratch_types=[
            pltpu.VMEM((per_w,), jnp.int32),
            pltpu.VMEM((per_w, N_EMBED), jnp.float32),
            pltpu.SemaphoreType.DMA,
        ],
    )
    def sk(rows_hbm, idx_hbm, out_hbm, idx_v, rows_v, sem):
        wid = lax.axis_index("s") * _NC + lax.axis_index("c")
        base = wid * per_w
        pltpu.sync_copy(idx_hbm.at[pl.ds(base, per_w)], idx_v)
        pltpu.sync_copy(rows_hbm.at[pl.ds(base, per_w)], rows_v)
        pltpu.async_copy(rows_v, out_hbm.at[idx_v], sem).wait()

    return sk(rows, idx)


def _gather_rows(table, idx):
    r_out = idx.shape[0]
    per_w = r_out // _NW
    ch = min(per_w, _CH)
    chunks = per_w // ch
    mesh = plsc.VectorSubcoreMesh(core_axis_name="c", subcore_axis_name="s")

    @functools.partial(
        pl.kernel,
        out_type=jax.ShapeDtypeStruct((r_out, N_EMBED), jnp.float32),
        mesh=mesh,
        scratch_types=[
            pltpu.VMEM((ch,), jnp.int32),
            pltpu.VMEM((ch, N_EMBED), jnp.float32),
            pltpu.SemaphoreType.DMA,
        ],
    )
    def gk(table_hbm, idx_hbm, out_hbm, idx_v, rows_v, sem):
        wid = lax.axis_index("s") * _NC + lax.axis_index("c")
        for c in range(chunks):
            base = wid * per_w + c * ch
            pltpu.sync_copy(idx_hbm.at[pl.ds(base, ch)], idx_v)
            pltpu.async_copy(table_hbm.at[idx_v], rows_v, sem).wait()
            pltpu.sync_copy(rows_v, out_hbm.at[pl.ds(base, ch)])

    return gk(table, idx)


HH = HIDDEN // 2


def _ffn_body(be_ref, used_ref, x_ref, w1a_ref, w1b_ref, b1_ref,
              w2a_ref, w2b_ref, b2_ref, o_ref):
    b = pl.program_id(0)

    @pl.when(b < used_ref[0])
    def _():
        x = x_ref[...]
        ha = jnp.dot(x, w1a_ref[0], preferred_element_type=jnp.float32)
        ha = jnp.maximum(ha + b1_ref[0, :, :HH], 0.0)
        hb = jnp.dot(x, w1b_ref[0], preferred_element_type=jnp.float32)
        hb = jnp.maximum(hb + b1_ref[0, :, HH:], 0.0)
        o_ref[...] = (
            jnp.dot(ha, w2a_ref[0], preferred_element_type=jnp.float32)
            + jnp.dot(hb, w2b_ref[0], preferred_element_type=jnp.float32)
            + b2_ref[0])


def _ffn(xp, w1, b1, w2, b2, block_expert, used):
    grid_spec = pltpu.PrefetchScalarGridSpec(
        num_scalar_prefetch=2,
        grid=(NB,),
        in_specs=[
            pl.BlockSpec((TB, N_EMBED),
                         lambda b, be, u: (jnp.minimum(b, u[0] - 1), 0)),
            pl.BlockSpec((1, N_EMBED, HH), lambda b, be, u: (be[b], 0, 0)),
            pl.BlockSpec((1, N_EMBED, HH), lambda b, be, u: (be[b], 0, 1)),
            pl.BlockSpec((1, 1, HIDDEN), lambda b, be, u: (be[b], 0, 0)),
            pl.BlockSpec((1, HH, N_EMBED), lambda b, be, u: (be[b], 0, 0)),
            pl.BlockSpec((1, HH, N_EMBED), lambda b, be, u: (be[b], 1, 0)),
            pl.BlockSpec((1, 1, N_EMBED), lambda b, be, u: (be[b], 0, 0)),
        ],
        out_specs=pl.BlockSpec((TB, N_EMBED),
                               lambda b, be, u: (jnp.minimum(b, u[0] - 1), 0)),
    )
    return pl.pallas_call(
        _ffn_body,
        grid_spec=grid_spec,
        out_shape=jax.ShapeDtypeStruct((NPAD, N_EMBED), jnp.float32),
        compiler_params=pltpu.CompilerParams(
            dimension_semantics=("arbitrary",),
        ),
    )(block_expert, used, xp, w1, w1,
      b1.reshape(E, 1, HIDDEN), w2, w2, b2.reshape(E, 1, N_EMBED))


def kernel(x, router_w, router_b, noise_w, noise_b,
           expert_w1, expert_b1, expert_w2, expert_b2):
    x2 = x.reshape(S, N_EMBED)
    wcat = jnp.concatenate([router_w, noise_w], axis=1)
    bcat = jnp.concatenate([router_b, noise_b]).reshape(1, 2 * E)
    noise = jax.random.normal(
        jax.random.key(42), (1, S, E), dtype=jnp.float32).reshape(S, E)

    dest2, be2, used2 = _router(x2, wcat, bcat, noise)
    dest = dest2[:, 0]
    xp = _scatter_rows(x2, dest, NPAD)
    yp = _ffn(xp, expert_w1, expert_b1, expert_w2, expert_b2,
              be2[:, 0], used2[0])
    y = _gather_rows(yp, dest)
    return y.reshape(x.shape)


# speedup vs baseline: 1.0267x; 1.0196x over previous
import functools

import jax
import jax.numpy as jnp
from jax import lax
from jax.experimental import pallas as pl
from jax.experimental.pallas import tpu as pltpu
from jax.experimental.pallas import tpu_sc as plsc

N_EMBED = 768
HIDDEN = 3072
E = 64
S = 2048
TB = 64
NB = S // TB + E
NPAD = NB * TB
GS = 512

_NC = 2
_NS = 16
_NW = _NC * _NS
_CH = 96


def _router_body(x_ref, wcat_ref, bcat_ref, noise_ref,
                 dest_ref, be_ref, used_ref):
    x = x_ref[...]
    both = jnp.dot(x, wcat_ref[...], preferred_element_type=jnp.float32)
    both = both + bcat_ref[...]
    logits = both[:, :E]
    nlog = both[:, E:]
    noisy = logits + noise_ref[...] * jax.nn.softplus(nlog)
    m = jnp.max(noisy, axis=1, keepdims=True)
    col = lax.broadcasted_iota(jnp.int32, (S, E), 1)
    e_idx = jnp.min(jnp.where(noisy >= m, col, E), axis=1, keepdims=True)

    oh = (col == e_idx).astype(jnp.float32)
    counts = jnp.sum(oh, axis=0, keepdims=True)
    nblk = jnp.floor((counts + (TB - 1)) * (1.0 / TB))
    ei = lax.broadcasted_iota(jnp.int32, (E, E), 0)
    ej = lax.broadcasted_iota(jnp.int32, (E, E), 1)
    upper = (ei <= ej).astype(jnp.float32)
    blk_end = jnp.dot(nblk, upper, preferred_element_type=jnp.float32)
    pad_off = (blk_end - nblk) * TB
    used_f = blk_end[:, E - 1:E]

    gi = lax.broadcasted_iota(jnp.int32, (GS, GS), 0)
    gj = lax.broadcasted_iota(jnp.int32, (GS, GS), 1)
    tri = (gj <= gi).astype(jnp.float32)
    base = jnp.zeros((1, E), jnp.float32)
    for g in range(S // GS):
        oh_g = oh[g * GS:(g + 1) * GS, :]
        cum_g = jnp.dot(tri, oh_g, preferred_element_type=jnp.float32) + base
        dest_g = jnp.sum((cum_g - 1.0 + pad_off) * oh_g, axis=1, keepdims=True)
        dest_ref[g * GS:(g + 1) * GS, :] = dest_g.astype(jnp.int32)
        base = cum_g[GS - 1:GS, :]

    bi = lax.broadcasted_iota(jnp.int32, (NB, 1), 0).astype(jnp.float32)
    be_raw = jnp.sum((blk_end <= bi).astype(jnp.float32), axis=1, keepdims=True)
    be_clamped = jnp.minimum(be_raw, E - 1)
    last_e = jnp.sum(jnp.where(bi == used_f - 1.0, be_clamped, 0.0),
                     axis=0, keepdims=True)
    be = jnp.where(bi < used_f, be_clamped, last_e)
    be_ref[...] = be.astype(jnp.int32)
    used_ref[...] = used_f.astype(jnp.int32)


def _router(x2, wcat, bcat, noise):
    return pl.pallas_call(
        _router_body,
        out_shape=(
            jax.ShapeDtypeStruct((S, 1), jnp.int32),
            jax.ShapeDtypeStruct((NB, 1), jnp.int32),
            jax.ShapeDtypeStruct((1, 1), jnp.int32),
        ),
    )(x2, wcat, bcat, noise)


def _scatter_rows(rows, idx, n_out):
    r_in = idx.shape[0]
    per_w = r_in // _NW
    mesh = plsc.VectorSubcoreMesh(core_axis_name="c", subcore_axis_name="s")

    @functools.partial(
        pl.kernel,
        out_type=jax.ShapeDtypeStruct((n_out, N_EMBED), jnp.float32),
        mesh=mesh,
        scratch_types=[
            pltpu.VMEM((per_w,), jnp.int32),
            pltpu.VMEM((per_w, N_EMBED), jnp.float32),
            pltpu.SemaphoreType.DMA,
        ],
    )
    def sk(rows_hbm, idx_hbm, out_hbm, idx_v, rows_v, sem):
        wid = lax.axis_index("s") * _NC + lax.axis_index("c")
        base = wid * per_w
        pltpu.sync_copy(idx_hbm.at[pl.ds(base, per_w)], idx_v)
        pltpu.sync_copy(rows_hbm.at[pl.ds(base, per_w)], rows_v)
        pltpu.async_copy(rows_v, out_hbm.at[idx_v], sem).wait()

    return sk(rows, idx)


def _gather_rows(table, idx):
    r_out = idx.shape[0]
    per_w = r_out // _NW
    ch = min(per_w, _CH)
    chunks = per_w // ch
    mesh = plsc.VectorSubcoreMesh(core_axis_name="c", subcore_axis_name="s")

    @functools.partial(
        pl.kernel,
        out_type=jax.ShapeDtypeStruct((r_out, N_EMBED), jnp.float32),
        mesh=mesh,
        scratch_types=[
            pltpu.VMEM((ch,), jnp.int32),
            pltpu.VMEM((ch, N_EMBED), jnp.float32),
            pltpu.SemaphoreType.DMA,
        ],
    )
    def gk(table_hbm, idx_hbm, out_hbm, idx_v, rows_v, sem):
        wid = lax.axis_index("s") * _NC + lax.axis_index("c")
        for c in range(chunks):
            base = wid * per_w + c * ch
            pltpu.sync_copy(idx_hbm.at[pl.ds(base, ch)], idx_v)
            pltpu.async_copy(table_hbm.at[idx_v], rows_v, sem).wait()
            pltpu.sync_copy(rows_v, out_hbm.at[pl.ds(base, ch)])

    return gk(table, idx)


HH = HIDDEN // 2


def _ffn_body(be_ref, used_ref, x_ref, w1a_ref, w1b_ref, b1_ref,
              w2a_ref, w2b_ref, b2_ref, o_ref):
    b = pl.program_id(0)

    @pl.when(b < used_ref[0])
    def _():
        x = x_ref[...]
        ha = jnp.dot(x, w1a_ref[0], preferred_element_type=jnp.float32)
        ha = jnp.maximum(ha + b1_ref[0, :, :HH], 0.0)
        hb = jnp.dot(x, w1b_ref[0], preferred_element_type=jnp.float32)
        hb = jnp.maximum(hb + b1_ref[0, :, HH:], 0.0)
        o_ref[...] = (
            jnp.dot(ha, w2a_ref[0], preferred_element_type=jnp.float32)
            + jnp.dot(hb, w2b_ref[0], preferred_element_type=jnp.float32)
            + b2_ref[0])


def _ffn(xp, w1, b1, w2, b2, block_expert, used):
    grid_spec = pltpu.PrefetchScalarGridSpec(
        num_scalar_prefetch=2,
        grid=(NB,),
        in_specs=[
            pl.BlockSpec((TB, N_EMBED),
                         lambda b, be, u: (jnp.minimum(b, u[0] - 1), 0)),
            pl.BlockSpec((1, N_EMBED, HH), lambda b, be, u: (be[b], 0, 0)),
            pl.BlockSpec((1, N_EMBED, HH), lambda b, be, u: (be[b], 0, 1)),
            pl.BlockSpec((1, 1, HIDDEN), lambda b, be, u: (be[b], 0, 0)),
            pl.BlockSpec((1, HH, N_EMBED), lambda b, be, u: (be[b], 0, 0)),
            pl.BlockSpec((1, HH, N_EMBED), lambda b, be, u: (be[b], 1, 0)),
            pl.BlockSpec((1, 1, N_EMBED), lambda b, be, u: (be[b], 0, 0)),
        ],
        out_specs=pl.BlockSpec((TB, N_EMBED),
                               lambda b, be, u: (jnp.minimum(b, u[0] - 1), 0)),
    )
    return pl.pallas_call(
        _ffn_body,
        grid_spec=grid_spec,
        out_shape=jax.ShapeDtypeStruct((NPAD, N_EMBED), jnp.float32),
        compiler_params=pltpu.CompilerParams(
            dimension_semantics=("arbitrary",),
        ),
    )(block_expert, used, xp, w1, w1,
      b1.reshape(E, 1, HIDDEN), w2, w2, b2.reshape(E, 1, N_EMBED))


def kernel(x, router_w, router_b, noise_w, noise_b,
           expert_w1, expert_b1, expert_w2, expert_b2):
    x2 = x.reshape(S, N_EMBED)
    wcat = jnp.concatenate([router_w, noise_w], axis=1)
    bcat = jnp.concatenate([router_b, noise_b]).reshape(1, 2 * E)
    with jax.ensure_compile_time_eval():
        noise = jax.random.normal(
            jax.random.key(42), (1, S, E), dtype=jnp.float32).reshape(S, E)

    dest2, be2, used2 = _router(x2, wcat, bcat, noise)
    dest = dest2[:, 0]
    xp = _scatter_rows(x2, dest, NPAD)
    yp = _ffn(xp, expert_w1, expert_b1, expert_w2, expert_b2,
              be2[:, 0], used2[0])
    y = _gather_rows(yp, dest)
    return y.reshape(x.shape)
